# baseline (speedup 1.0000x reference)
import functools

import jax
import jax.numpy as jnp
from jax import lax
from jax.experimental import pallas as pl
from jax.experimental.pallas import tpu as pltpu
from jax.experimental.pallas import tpu_sc as plsc

N_NODES = 10000
N_EDGES = 320000
ACC_ROWS = 10240
D = 128
GRP = 128
N_TILES = 32
K_PER_TILE = 80
NBUF = 4
N_GROUPS = K_PER_TILE * N_TILES
E_PAD = N_GROUPS * GRP
RPT = ACC_ROWS // 16

_MESH = plsc.VectorSubcoreMesh(core_axis_name="c", subcore_axis_name="s")


def _make_agg(W):
  @functools.partial(
      pl.kernel, mesh=_MESH,
      out_type=jax.ShapeDtypeStruct((2 * ACC_ROWS, W), jnp.float32),
      scratch_types=[
          pltpu.VMEM((K_PER_TILE // 2, GRP), jnp.int32),
          pltpu.VMEM((K_PER_TILE // 2, GRP), jnp.int32),
          pltpu.VMEM((GRP, W), jnp.float32),
          pltpu.VMEM((GRP, W), jnp.float32),
          pltpu.VMEM_SHARED((ACC_ROWS, W), jnp.float32),
          pltpu.SemaphoreType.DMA,
          pltpu.SemaphoreType.DMA,
      ],
      compiler_params=pltpu.CompilerParams(use_tc_tiling_on_sc=(W == 128)))
  def _agg(u_hbm, z_hbm, srcg_hbm, dstg_hbm, p_hbm,
           srcv, dstv, rows0, rows1, acc, semg0, semg1):
    c = lax.axis_index("c")
    s = lax.axis_index("s")
    tile = c * 16 + s
    rows = (rows0, rows1)
    sems = (semg0, semg1)
    KH = K_PER_TILE // 2

    @pl.when(c == 0)
    def _():
      pltpu.sync_copy(u_hbm.at[pl.ds(s * RPT, RPT)],
                      acc.at[pl.ds(s * RPT, RPT)])

    @pl.when(c == 1)
    def _():
      pltpu.sync_copy(z_hbm.at[pl.ds(s * RPT, RPT)],
                      acc.at[pl.ds(s * RPT, RPT)])
    plsc.subcore_barrier()

    def gather(g, i):
      return pltpu.make_async_copy(u_hbm.at[srcv.at[g]], rows[i], sems[i])

    def scat(g, i):
      pltpu.sync_copy(rows[i], acc.at[dstv.at[g]], add=True)

    for h in range(2):
      pltpu.sync_copy(srcg_hbm.at[pl.ds(tile * K_PER_TILE + h * KH, KH)],
                      srcv)
      pltpu.sync_copy(dstg_hbm.at[pl.ds(tile * K_PER_TILE + h * KH, KH)],
                      dstv)
      gather(0, 0).start()

      def body(t, carry):
        g0 = 2 * t
        gather(g0 + 1, 1).start()
        gather(g0, 0).wait()
        scat(g0, 0)
        gather(g0 + 2, 0).start()
        gather(g0 + 1, 1).wait()
        scat(g0 + 1, 1)
        return carry

      lax.fori_loop(0, KH // 2 - 1, body, 0)
      g0 = KH - 2
      gather(g0 + 1, 1).start()
      gather(g0, 0).wait()
      scat(g0, 0)
      gather(g0 + 1, 1).wait()
      scat(g0 + 1, 1)

    plsc.subcore_barrier()
    pltpu.sync_copy(acc.at[pl.ds(s * RPT, RPT)],
                    p_hbm.at[pl.ds(c * ACC_ROWS + s * RPT, RPT)])

  return _agg


def _mm(a, w):
  return jnp.dot(a.astype(jnp.bfloat16), w.astype(jnp.bfloat16),
                 preferred_element_type=jnp.float32)


def _tc1(degp_ref, x_ref, w1_ref, dinv_ref, u1_ref):
  deg = degp_ref[:ACC_ROWS, 0:1] + degp_ref[ACC_ROWS:, 0:1]
  dinv = lax.rsqrt(deg)
  dinv_ref[...] = dinv
  u1_ref[...] = _mm(x_ref[...], w1_ref[...]) * dinv


def _bn(t):
  mask = lax.broadcasted_iota(jnp.int32, (ACC_ROWS, 1), 0) < N_NODES
  tm = jnp.where(mask, t, 0.0)
  mean = jnp.sum(tm, axis=0, keepdims=True) * (1.0 / N_NODES)
  cen = t - mean
  var = jnp.sum(jnp.where(mask, cen * cen, 0.0), axis=0,
                keepdims=True) * (1.0 / N_NODES)
  return cen * lax.rsqrt(var + 1e-5)


def _tc_mid(p_ref, dinv_ref, b_ref, g_ref, be_ref, w_ref, u_ref):
  dv = dinv_ref[...]
  t = (p_ref[:ACC_ROWS] + p_ref[ACC_ROWS:]) * dv + b_ref[...]
  y = jnp.maximum(_bn(t) * g_ref[...] + be_ref[...], 0.0)
  u_ref[...] = _mm(y, w_ref[...]) * dv


def _tc_head(p_ref, dinv_ref, b_ref, g_ref, be_ref,
             fw1_ref, fb1_ref, fw2_ref, fb2_ref, fw3_ref, fb3_ref,
             fw4_ref, fb4_ref, out_ref):
  t = (p_ref[:ACC_ROWS] + p_ref[ACC_ROWS:]) * dinv_ref[...] + b_ref[...]
  h = jnp.maximum(_bn(t) * g_ref[...] + be_ref[...], 0.0)
  h = jnp.maximum(_mm(h, fw1_ref[...]) + fb1_ref[...], 0.0)
  h = jnp.maximum(_mm(h, fw2_ref[...]) + fb2_ref[...], 0.0)
  h = jnp.maximum(_mm(h, fw3_ref[...]) + fb3_ref[...], 0.0)
  out_ref[...] = _mm(h, fw4_ref[...]) + fb4_ref[...]


def _colpad(w, n):
  return jnp.pad(w, ((0, 0), (0, n - w.shape[1])))


def kernel(x, W1, b1, g1, be1, W2, b2, g2, be2, W3, b3, g3, be3,
           fw1, fb1, fw2, fb2, fw3, fb3, fw4, fb4, edge_index):
  f32 = jnp.float32

  pad = E_PAD - N_EDGES
  ar = jnp.arange(pad, dtype=jnp.int32)
  pads = jnp.stack([ar & 8191, N_NODES + (ar & 127)])
  eg = jnp.concatenate([edge_index, pads], axis=1).reshape(2, -1, GRP)
  src_g = eg[0]
  dst_g = eg[1]

  x_pad = jnp.pad(x, ((0, ACC_ROWS - N_NODES), (0, 0)))
  ones8 = jnp.ones((ACC_ROWS, 8), f32)
  z8 = jnp.zeros((ACC_ROWS, 8), f32)
  z32 = jnp.zeros((ACC_ROWS, 32), f32)
  z64 = jnp.zeros((ACC_ROWS, 64), f32)
  z128 = jnp.zeros((ACC_ROWS, D), f32)


  degp = _make_agg(8)(ones8, z8, src_g, dst_g)

  dinv, u1 = pl.pallas_call(
      _tc1,
      out_shape=(jax.ShapeDtypeStruct((ACC_ROWS, 1), f32),
                 jax.ShapeDtypeStruct((ACC_ROWS, 32), f32)),
  )(degp, x_pad, W1)

  ap = _make_agg(32)(u1, z32, src_g, dst_g)
  u2 = pl.pallas_call(
      _tc_mid,
      out_shape=jax.ShapeDtypeStruct((ACC_ROWS, 64), f32),
  )(ap, dinv, b1.reshape(1, -1), g1.reshape(1, -1), be1.reshape(1, -1),
    W2)

  bp = _make_agg(64)(u2, z64, src_g, dst_g)
  u3 = pl.pallas_call(
      _tc_mid,
      out_shape=jax.ShapeDtypeStruct((ACC_ROWS, 128), f32),
  )(bp, dinv, b2.reshape(1, -1), g2.reshape(1, -1), be2.reshape(1, -1),
    W3)

  cp = _make_agg(128)(u3, z128, src_g, dst_g)
  out = pl.pallas_call(
      _tc_head,
      out_shape=jax.ShapeDtypeStruct((ACC_ROWS, 40), f32),
  )(cp, dinv, b3.reshape(1, -1), g3.reshape(1, -1), be3.reshape(1, -1),
    fw1, fb1.reshape(1, -1), fw2, fb2.reshape(1, -1),
    fw3, fb3.reshape(1, -1), fw4, fb4.reshape(1, -1))

  return out[:N_NODES]


# speedup vs baseline: 1.1244x; 1.1244x over previous
import functools

import jax
import jax.numpy as jnp
from jax import lax
from jax.experimental import pallas as pl
from jax.experimental.pallas import tpu as pltpu
from jax.experimental.pallas import tpu_sc as plsc

N_NODES = 10000
N_EDGES = 320000
ACC_ROWS = 10240
D = 128
GRP = 128
N_TILES = 32
K_PER_TILE = 80
NBUF = 4
N_GROUPS = K_PER_TILE * N_TILES
E_PAD = N_GROUPS * GRP
RPT = ACC_ROWS // 16

_MESH = plsc.VectorSubcoreMesh(core_axis_name="c", subcore_axis_name="s")


@functools.partial(
    pl.kernel, mesh=_MESH,
    out_type=jax.ShapeDtypeStruct((2 * ACC_ROWS, 8), jnp.float32),
    scratch_types=[
        pltpu.VMEM((K_PER_TILE, GRP), jnp.int32),
        pltpu.VMEM((GRP, 8), jnp.float32),
        pltpu.VMEM_SHARED((ACC_ROWS, 8), jnp.float32),
        pltpu.SemaphoreType.DMA,
    ])
def _deg(ones_hbm, z_hbm, dstg_hbm, p_hbm, dstv, onesv, acc, sem):
  c = lax.axis_index("c")
  s = lax.axis_index("s")
  tile = c * 16 + s

  @pl.when(c == 0)
  def _():
    pltpu.sync_copy(ones_hbm.at[pl.ds(s * RPT, RPT)],
                    acc.at[pl.ds(s * RPT, RPT)])

  @pl.when(c == 1)
  def _():
    pltpu.sync_copy(z_hbm.at[pl.ds(s * RPT, RPT)],
                    acc.at[pl.ds(s * RPT, RPT)])

  pltpu.sync_copy(ones_hbm.at[pl.ds(0, GRP)], onesv)
  pltpu.sync_copy(dstg_hbm.at[pl.ds(tile * K_PER_TILE, K_PER_TILE)], dstv)
  plsc.subcore_barrier()

  def scat(g):
    return pltpu.make_async_copy(onesv, acc.at[dstv.at[g]], sem)

  def body(j, carry):
    scat(j).start(add=True)

    @pl.when(j >= 4)
    def _():
      scat(j - 4).wait()
    return carry

  lax.fori_loop(0, K_PER_TILE, body, 0)
  for g in range(K_PER_TILE - 4, K_PER_TILE):
    scat(g).wait()
  plsc.subcore_barrier()
  pltpu.sync_copy(acc.at[pl.ds(s * RPT, RPT)],
                  p_hbm.at[pl.ds(c * ACC_ROWS + s * RPT, RPT)])


def _make_agg(W):
  @functools.partial(
      pl.kernel, mesh=_MESH,
      out_type=jax.ShapeDtypeStruct((2 * ACC_ROWS, W), jnp.float32),
      scratch_types=(
          [pltpu.VMEM((K_PER_TILE // 2, GRP), jnp.int32),
           pltpu.VMEM((K_PER_TILE // 2, GRP), jnp.int32)] +
          [pltpu.VMEM((GRP, W), jnp.float32)] * (2 if W == 128 else 4) +
          [pltpu.VMEM_SHARED((ACC_ROWS, W), jnp.float32)] +
          [pltpu.SemaphoreType.DMA] * (2 if W == 128 else 8)),
      compiler_params=pltpu.CompilerParams(use_tc_tiling_on_sc=(W == 128)))
  def _agg(u_hbm, z_hbm, srcg_hbm, dstg_hbm, p_hbm, srcv, dstv, *rest):
    if W == 128:
      rows = rest[0:2]
      acc = rest[2]
      sems = rest[3:5]
      semsc = ()
    else:
      rows = rest[0:4]
      acc = rest[4]
      sems = rest[5:9]
      semsc = rest[9:13]
    c = lax.axis_index("c")
    s = lax.axis_index("s")
    tile = c * 16 + s
    KH = K_PER_TILE // 2

    @pl.when(c == 0)
    def _():
      pltpu.sync_copy(u_hbm.at[pl.ds(s * RPT, RPT)],
                      acc.at[pl.ds(s * RPT, RPT)])

    @pl.when(c == 1)
    def _():
      pltpu.sync_copy(z_hbm.at[pl.ds(s * RPT, RPT)],
                      acc.at[pl.ds(s * RPT, RPT)])
    plsc.subcore_barrier()

    def gather(g, i):
      return pltpu.make_async_copy(u_hbm.at[srcv.at[g]], rows[i], sems[i])

    def scat(g, i):
      pltpu.sync_copy(rows[i], acc.at[dstv.at[g]], add=True)

    def ascat(g, i):
      return pltpu.make_async_copy(rows[i], acc.at[dstv.at[g]], semsc[i])

    for h in range(2):
      pltpu.sync_copy(srcg_hbm.at[pl.ds(tile * K_PER_TILE + h * KH, KH)],
                      srcv)
      pltpu.sync_copy(dstg_hbm.at[pl.ds(tile * K_PER_TILE + h * KH, KH)],
                      dstv)
      if W < 128:
        gather(0, 0).start()
        gather(1, 1).start()

        def body(t, carry):
          for i in range(4):
            g = 4 * t + i
            gather(g, i).wait()
            ascat(g, i).start(add=True)

            @pl.when(g >= 2)
            def _():
              ascat(g - 2, (i - 2) % 4).wait()

            @pl.when(g + 2 < KH)
            def _():
              gather(g + 2, (i + 2) % 4).start()
          return carry

        lax.fori_loop(0, KH // 4, body, 0)
        ascat(KH - 2, (KH - 2) % 4).wait()
        ascat(KH - 1, (KH - 1) % 4).wait()
      else:
        gather(0, 0).start()

        def body(t, carry):
          g0 = 2 * t
          gather(g0 + 1, 1).start()
          gather(g0, 0).wait()
          scat(g0, 0)
          gather(g0 + 2, 0).start()
          gather(g0 + 1, 1).wait()
          scat(g0 + 1, 1)
          return carry

        lax.fori_loop(0, KH // 2 - 1, body, 0)
        g0 = KH - 2
        gather(g0 + 1, 1).start()
        gather(g0, 0).wait()
        scat(g0, 0)
        gather(g0 + 1, 1).wait()
        scat(g0 + 1, 1)

    plsc.subcore_barrier()
    pltpu.sync_copy(acc.at[pl.ds(s * RPT, RPT)],
                    p_hbm.at[pl.ds(c * ACC_ROWS + s * RPT, RPT)])

  return _agg


def _mm(a, w):
  return jnp.dot(a.astype(jnp.bfloat16), w.astype(jnp.bfloat16),
                 preferred_element_type=jnp.float32)


def _tc1(degp_ref, x_ref, w1_ref, dinv_ref, u1_ref):
  deg = degp_ref[:ACC_ROWS, 0:1] + degp_ref[ACC_ROWS:, 0:1]
  dinv = lax.rsqrt(deg)
  dinv_ref[...] = dinv
  u1_ref[...] = _mm(x_ref[...], w1_ref[...]) * dinv


def _bn(t):
  mask = lax.broadcasted_iota(jnp.int32, (ACC_ROWS, 1), 0) < N_NODES
  tm = jnp.where(mask, t, 0.0)
  mean = jnp.sum(tm, axis=0, keepdims=True) * (1.0 / N_NODES)
  cen = t - mean
  var = jnp.sum(jnp.where(mask, cen * cen, 0.0), axis=0,
                keepdims=True) * (1.0 / N_NODES)
  return cen * lax.rsqrt(var + 1e-5)


def _tc_mid(p_ref, dinv_ref, b_ref, g_ref, be_ref, w_ref, u_ref):
  dv = dinv_ref[...]
  t = (p_ref[:ACC_ROWS] + p_ref[ACC_ROWS:]) * dv + b_ref[...]
  y = jnp.maximum(_bn(t) * g_ref[...] + be_ref[...], 0.0)
  u_ref[...] = _mm(y, w_ref[...]) * dv


def _tc_head(p_ref, dinv_ref, b_ref, g_ref, be_ref,
             fw1_ref, fb1_ref, fw2_ref, fb2_ref, fw3_ref, fb3_ref,
             fw4_ref, fb4_ref, out_ref):
  t = (p_ref[:ACC_ROWS] + p_ref[ACC_ROWS:]) * dinv_ref[...] + b_ref[...]
  h = jnp.maximum(_bn(t) * g_ref[...] + be_ref[...], 0.0)
  h = jnp.maximum(_mm(h, fw1_ref[...]) + fb1_ref[...], 0.0)
  h = jnp.maximum(_mm(h, fw2_ref[...]) + fb2_ref[...], 0.0)
  h = jnp.maximum(_mm(h, fw3_ref[...]) + fb3_ref[...], 0.0)
  out_ref[...] = (_mm(h, fw4_ref[...]) + fb4_ref[...])[:N_NODES]


def _colpad(w, n):
  return jnp.pad(w, ((0, 0), (0, n - w.shape[1])))


def kernel(x, W1, b1, g1, be1, W2, b2, g2, be2, W3, b3, g3, be3,
           fw1, fb1, fw2, fb2, fw3, fb3, fw4, fb4, edge_index):
  f32 = jnp.float32

  pad = E_PAD - N_EDGES
  ar = jnp.arange(pad, dtype=jnp.int32)
  pads = jnp.stack([ar & 8191, N_NODES + (ar & 127)])
  eg = jnp.concatenate([edge_index, pads], axis=1).reshape(2, -1, GRP)
  src_g = eg[0]
  dst_g = eg[1]

  x_pad = jnp.pad(x, ((0, ACC_ROWS - N_NODES), (0, 0)))
  ones8 = jnp.ones((ACC_ROWS, 8), f32)
  z8 = jnp.zeros((ACC_ROWS, 8), f32)
  z32 = jnp.zeros((ACC_ROWS, 32), f32)
  z64 = jnp.zeros((ACC_ROWS, 64), f32)
  z128 = jnp.zeros((ACC_ROWS, D), f32)


  degp = _deg(ones8, z8, dst_g)

  dinv, u1 = pl.pallas_call(
      _tc1,
      out_shape=(jax.ShapeDtypeStruct((ACC_ROWS, 1), f32),
                 jax.ShapeDtypeStruct((ACC_ROWS, 32), f32)),
  )(degp, x_pad, W1)

  ap = _make_agg(32)(u1, z32, src_g, dst_g)
  u2 = pl.pallas_call(
      _tc_mid,
      out_shape=jax.ShapeDtypeStruct((ACC_ROWS, 64), f32),
  )(ap, dinv, b1.reshape(1, -1), g1.reshape(1, -1), be1.reshape(1, -1),
    W2)

  bp = _make_agg(64)(u2, z64, src_g, dst_g)
  u3 = pl.pallas_call(
      _tc_mid,
      out_shape=jax.ShapeDtypeStruct((ACC_ROWS, 128), f32),
  )(bp, dinv, b2.reshape(1, -1), g2.reshape(1, -1), be2.reshape(1, -1),
    W3)

  cp = _make_agg(128)(u3, z128, src_g, dst_g)
  out = pl.pallas_call(
      _tc_head,
      out_shape=jax.ShapeDtypeStruct((N_NODES, 40), f32),
  )(cp, dinv, b3.reshape(1, -1), g3.reshape(1, -1), be3.reshape(1, -1),
    fw1, fb1.reshape(1, -1), fw2, fb2.reshape(1, -1),
    fw3, fb3.reshape(1, -1), fw4, fb4.reshape(1, -1))

  return out
